# baseline (speedup 1.0000x reference)
import functools

import jax
import jax.numpy as jnp
from jax import lax
from jax.experimental import pallas as pl
from jax.experimental.pallas import tpu as pltpu
from jax.experimental.pallas import tpu_sc as plsc

N_NODES = 100000
NUM_EMB = 21
HIDDEN = 128

NC = 2
NS = 16
NW = NC * NS

CHUNK = 128
CHUNKS = 25
PER_W = CHUNK * CHUNKS
N_PAD = NW * PER_W

NBUF = 2

_mesh = plsc.VectorSubcoreMesh(core_axis_name="c", subcore_axis_name="s")


@functools.partial(
    pl.kernel,
    mesh=_mesh,
    out_type=jax.ShapeDtypeStruct((N_PAD, HIDDEN), jnp.float32),
    scratch_types=[
        pltpu.VMEM_SHARED((NUM_EMB, HIDDEN), jnp.float32),
        pltpu.VMEM((CHUNKS, CHUNK), jnp.int32),
        pltpu.VMEM((NBUF, CHUNK, HIDDEN), jnp.float32),
        pltpu.SemaphoreType.DMA((NBUF,)),
        pltpu.SemaphoreType.DMA((NBUF,)),
    ],
)
def _emb_lookup(idx_hbm, table_hbm, out_hbm, table_sh, idx_v, rows_v, gsem, wsem):
    sid = lax.axis_index("s")
    wid = sid * NC + lax.axis_index("c")
    base = wid * PER_W

    @pl.when(sid == 0)
    def _stage_table():
        pltpu.sync_copy(table_hbm, table_sh)

    pltpu.sync_copy(idx_hbm.at[wid], idx_v)
    plsc.subcore_barrier()

    gathers = [None] * CHUNKS
    writes = [None] * CHUNKS
    for c in range(CHUNKS):
        b = c % NBUF
        if c >= NBUF:
            writes[c - NBUF].wait()
        gathers[c] = pltpu.async_copy(
            table_sh.at[idx_v.at[c]], rows_v.at[b], gsem.at[b])
        if c >= 1:
            pb = (c - 1) % NBUF
            gathers[c - 1].wait()
            writes[c - 1] = pltpu.async_copy(
                rows_v.at[pb], out_hbm.at[pl.ds(base + (c - 1) * CHUNK, CHUNK)],
                wsem.at[pb])
    gathers[CHUNKS - 1].wait()
    writes[CHUNKS - 1] = pltpu.async_copy(
        rows_v.at[(CHUNKS - 1) % NBUF],
        out_hbm.at[pl.ds(base + (CHUNKS - 1) * CHUNK, CHUNK)],
        wsem.at[(CHUNKS - 1) % NBUF])
    for c in range(CHUNKS - NBUF, CHUNKS):
        writes[c].wait()


def kernel(x, emb_weight):
    idx = jnp.pad(x.astype(jnp.int32), (0, N_PAD - N_NODES))
    idx = idx.reshape(NW, CHUNKS, CHUNK)
    out = _emb_lookup(idx, emb_weight)
    return out[:N_NODES]
